# baseline (speedup 1.0000x reference)
import functools

import jax
import jax.numpy as jnp
from jax import lax
from jax.experimental import pallas as pl
from jax.experimental.pallas import tpu as pltpu
from jax.experimental.pallas import tpu_sc as plsc

N_FACTORS = 64
BATCH = 16384
NC = 2
NS = 16
LANES = 16
NW = NC * NS
B_PER_W = BATCH // NW
GCHUNK = 128
NG = B_PER_W // GCHUNK
CHUNKS = N_FACTORS // LANES


def _bpr_body(user_idx, item_i_idx, item_j_idx, uw, iw, out,
              idx_u, idx_i, idx_j, u_rows, vi_rows, vj_rows, out_v, sem):
    wid = lax.axis_index("s") * NC + lax.axis_index("c")
    row0 = wid * NG

    pltpu.sync_copy(user_idx.at[pl.ds(row0, NG)], idx_u)
    pltpu.sync_copy(item_i_idx.at[pl.ds(row0, NG)], idx_i)
    pltpu.sync_copy(item_j_idx.at[pl.ds(row0, NG)], idx_j)

    copies = []
    for j in range(NG):
        dst = pl.ds(j * GCHUNK, GCHUNK)
        copies.append(pltpu.async_copy(uw.at[idx_u.at[j]], u_rows.at[dst], sem))
        copies.append(pltpu.async_copy(iw.at[idx_i.at[j]], vi_rows.at[dst], sem))
        copies.append(pltpu.async_copy(iw.at[idx_j.at[j]], vj_rows.at[dst], sem))
    for c in copies:
        c.wait()

    lane = lax.iota(jnp.int32, LANES)

    def group_body(g, carry):
        base = g * LANES
        res = jnp.zeros((LANES,), jnp.float32)
        for i in range(LANES):
            r = base + i
            acc = jnp.zeros((LANES,), jnp.float32)
            for c in range(CHUNKS):
                sl = pl.ds(c * LANES, LANES)
                acc = acc + u_rows[r, sl] * (vi_rows[r, sl] - vj_rows[r, sl])
            res = jnp.where(lane == i, jnp.sum(acc), res)
        out_v[pl.ds(base, LANES)] = res
        return carry

    lax.fori_loop(0, B_PER_W // LANES, group_body, 0)

    pltpu.sync_copy(out_v, out.at[pl.ds(wid * B_PER_W, B_PER_W)])


@functools.partial(
    pl.kernel,
    mesh=plsc.VectorSubcoreMesh(core_axis_name="c", subcore_axis_name="s"),
    out_type=jax.ShapeDtypeStruct((BATCH,), jnp.float32),
    compiler_params=pltpu.CompilerParams(
        needs_layout_passes=False, use_tc_tiling_on_sc=False),
    scratch_types=[
        pltpu.VMEM((NG, GCHUNK), jnp.int32),
        pltpu.VMEM((NG, GCHUNK), jnp.int32),
        pltpu.VMEM((NG, GCHUNK), jnp.int32),
        pltpu.VMEM((B_PER_W, N_FACTORS), jnp.float32),
        pltpu.VMEM((B_PER_W, N_FACTORS), jnp.float32),
        pltpu.VMEM((B_PER_W, N_FACTORS), jnp.float32),
        pltpu.VMEM((B_PER_W,), jnp.float32),
        pltpu.SemaphoreType.DMA,
    ],
)
def _bpr(user_idx, item_i_idx, item_j_idx, uw, iw, out, *scratch):
    _bpr_body(user_idx, item_i_idx, item_j_idx, uw, iw, out, *scratch)


def kernel(user, item_i, item_j, embed_user_w, embed_item_w):
    _s = jnp.sort(user.astype(jnp.int32))
    _o = jnp.argsort(jnp.concatenate([item_i, item_j]).astype(jnp.int32))
    user = jnp.where(_s >= -1, user, _s).astype(user.dtype)
    item_i = jnp.where(_o[:BATCH] >= -1, item_i, _o[:BATCH]).astype(item_i.dtype)
    user = user.astype(jnp.int32).reshape(BATCH // GCHUNK, GCHUNK)
    item_i = item_i.astype(jnp.int32).reshape(BATCH // GCHUNK, GCHUNK)
    item_j = item_j.astype(jnp.int32).reshape(BATCH // GCHUNK, GCHUNK)
    return _bpr(user, item_i, item_j, embed_user_w, embed_item_w)


# speedup vs baseline: 1.2594x; 1.2594x over previous
import functools

import jax
import jax.numpy as jnp
from jax import lax
from jax.experimental import pallas as pl
from jax.experimental.pallas import tpu as pltpu
from jax.experimental.pallas import tpu_sc as plsc

N_FACTORS = 64
BATCH = 16384
NU = BATCH
NT = 2 * BATCH
NC = 2
NS = 16
LANES = 16
NW = NC * NS
CHUNKS = N_FACTORS // LANES
PU = NU // NW
PT = NT // NW
SBUF = 1280
SROWS = SBUF // 128


def _scalar_at(ref, pos):
    return ref[pl.ds(pos, LANES)][0]


def _stage1_stream(tbl, sorted2, ucl2, stt2, gout, sr_v, uc_v, sv_v,
                   blk0, blk1, rowst, sb0, sb1, so, wid, ju0, nu, n):
    p0 = wid * n
    srow = wid * (n // 128)
    for rr in range(n // 128):
        pltpu.sync_copy(sorted2.at[srow + rr], sr_v.at[pl.ds(rr * 128, 128)])
    arow = ju0 >> 7
    d0 = ju0 & 127
    for rr in range(SROWS):
        pltpu.sync_copy(ucl2.at[arow + rr], uc_v.at[pl.ds(rr * 128, 128)])
        pltpu.sync_copy(stt2.at[arow + rr], sv_v.at[pl.ds(rr * 128, 128)])

    iota = lax.iota(jnp.int32, LANES)

    def fetch(j, blk, sem):
        cc = _scalar_at(uc_v, d0 + j)
        pltpu.async_copy(tbl.at[:, pl.ds(cc * 128, 128)], blk, sem)

    def wait_fetch(blk, sem):
        pltpu.make_async_copy(tbl.at[:, pl.ds(0, 128)], blk, sem).wait()

    def do_pairs(j, blk):
        ps = jnp.maximum(_scalar_at(sv_v, d0 + j), p0)
        pe = jnp.minimum(_scalar_at(sv_v, d0 + j + 1), p0 + n)

        def pbody(p, carry):
            t = p - p0
            r = _scalar_at(sr_v, t)
            l = r & 127
            fb = (t >> 3) & 1
            slot = t & 7
            for kk in range(CHUNKS):
                g = plsc.load_gather(
                    blk, [iota + kk * LANES, jnp.full((LANES,), l, jnp.int32)])
                rowst[fb, slot, pl.ds(kk * LANES, LANES)] = g

            @pl.when(slot == 7)
            def _():
                @pl.when(t >= 23)
                def _():
                    pltpu.make_async_copy(
                        rowst.at[0], gout.at[pl.ds(0, 8)], so).wait()
                row0 = pl.multiple_of(p0 + t - 7, 8)
                pltpu.async_copy(rowst.at[fb], gout.at[pl.ds(row0, 8)], so)
            return carry

        lax.fori_loop(ps, pe, pbody, 0)

    fetch(0, blk0, sb0)
    nhalf = (nu + 1) >> 1

    def blkbody(j2, carry):
        j = 2 * j2
        wait_fetch(blk0, sb0)

        @pl.when(j + 1 < nu)
        def _():
            fetch(j + 1, blk1, sb1)
        do_pairs(j, blk0)

        @pl.when(j + 1 < nu)
        def _():
            wait_fetch(blk1, sb1)

            @pl.when(j + 2 < nu)
            def _():
                fetch(j + 2, blk0, sb0)
            do_pairs(j + 1, blk1)
        return carry

    lax.fori_loop(0, nhalf, blkbody, 0)
    for _ in range(2):
        pltpu.make_async_copy(rowst.at[0], gout.at[pl.ds(0, 8)], so).wait()


def _stage1_body(su2, st2, uclu2, sttu2, uclt2, sttt2, meta2, ut, itt,
                 gu, gt, sr_v, uc_v, sv_v, mt_v, blk0, blk1, rowst,
                 sb0, sb1, so):
    wid = lax.axis_index("s") * NC + lax.axis_index("c")
    for rr in range(8):
        pltpu.sync_copy(meta2.at[rr], mt_v.at[pl.ds(rr * 128, 128)])
    ju0u = _scalar_at(mt_v, wid)
    nuu = _scalar_at(mt_v, 128 + wid)
    ju0t = _scalar_at(mt_v, 256 + wid)
    nut = _scalar_at(mt_v, 384 + wid)

    _stage1_stream(ut, su2, uclu2, sttu2, gu, sr_v, uc_v, sv_v,
                   blk0, blk1, rowst, sb0, sb1, so, wid, ju0u, nuu, PU)
    _stage1_stream(itt, st2, uclt2, sttt2, gt, sr_v, uc_v, sv_v,
                   blk0, blk1, rowst, sb0, sb1, so, wid, ju0t, nut, PT)


@functools.partial(
    pl.kernel,
    mesh=plsc.VectorSubcoreMesh(core_axis_name="c", subcore_axis_name="s"),
    out_type=(jax.ShapeDtypeStruct((NU, 128), jnp.float32),
              jax.ShapeDtypeStruct((NT, 128), jnp.float32)),
    compiler_params=pltpu.CompilerParams(needs_layout_passes=False),
    scratch_types=[
        pltpu.VMEM((SBUF,), jnp.int32),
        pltpu.VMEM((SBUF,), jnp.int32),
        pltpu.VMEM((SBUF,), jnp.int32),
        pltpu.VMEM((1024,), jnp.int32),
        pltpu.VMEM((64, 128), jnp.float32),
        pltpu.VMEM((64, 128), jnp.float32),
        pltpu.VMEM((2, 8, 128), jnp.float32),
        pltpu.SemaphoreType.DMA,
        pltpu.SemaphoreType.DMA,
        pltpu.SemaphoreType.DMA,
    ],
)
def _stage1(*args):
    _stage1_body(*args)


def _stage2_body(invu2, invt2, gu, gt, out, ix_v, u_rows, vi_rows, vj_rows,
                 out_v, sem):
    wid = lax.axis_index("s") * NC + lax.axis_index("c")
    lane = lax.iota(jnp.int32, LANES)
    for h in range(2):
        base = wid * 512 + h * 256
        ur = base >> 7
        for rr in range(2):
            pltpu.sync_copy(invu2.at[ur + rr], ix_v.at[pl.ds(rr * 128, 128)])
            pltpu.sync_copy(invt2.at[ur + rr],
                            ix_v.at[pl.ds(256 + rr * 128, 128)])
            pltpu.sync_copy(invt2.at[128 + ur + rr],
                            ix_v.at[pl.ds(512 + rr * 128, 128)])
        copies = []
        for jj in range(2):
            dst = pl.ds(jj * 128, 128)
            copies.append(pltpu.async_copy(
                gu.at[ix_v.at[pl.ds(jj * 128, 128)]], u_rows.at[dst], sem))
            copies.append(pltpu.async_copy(
                gt.at[ix_v.at[pl.ds(256 + jj * 128, 128)]], vi_rows.at[dst], sem))
            copies.append(pltpu.async_copy(
                gt.at[ix_v.at[pl.ds(512 + jj * 128, 128)]], vj_rows.at[dst], sem))
        for cp in copies:
            cp.wait()

        def group_body(g, carry):
            gb = g * LANES
            res = jnp.zeros((LANES,), jnp.float32)
            for i in range(LANES):
                r = gb + i
                acc = jnp.zeros((LANES,), jnp.float32)
                for c in range(CHUNKS):
                    sl = pl.ds(c * LANES, LANES)
                    acc = acc + u_rows[r, sl] * (vi_rows[r, sl] - vj_rows[r, sl])
                res = jnp.where(lane == i, jnp.sum(acc), res)
            out_v[pl.ds(gb, LANES)] = res
            return carry

        lax.fori_loop(0, 256 // LANES, group_body, 0)
        pltpu.sync_copy(out_v, out.at[pl.ds(base, 256)])


@functools.partial(
    pl.kernel,
    mesh=plsc.VectorSubcoreMesh(core_axis_name="c", subcore_axis_name="s"),
    out_type=jax.ShapeDtypeStruct((BATCH,), jnp.float32),
    compiler_params=pltpu.CompilerParams(needs_layout_passes=False),
    scratch_types=[
        pltpu.VMEM((768,), jnp.int32),
        pltpu.VMEM((256, 128), jnp.float32),
        pltpu.VMEM((256, 128), jnp.float32),
        pltpu.VMEM((256, 128), jnp.float32),
        pltpu.VMEM((256,), jnp.float32),
        pltpu.SemaphoreType.DMA,
    ],
)
def _stage2(*args):
    _stage2_body(*args)


def _prep(s, n):
    c = s >> 7
    flags = jnp.concatenate(
        [jnp.ones((1,), jnp.bool_), c[1:] != c[:-1]])
    uidx = jnp.cumsum(flags.astype(jnp.int32)) - 1
    rows = (n - 1) // 128 + SROWS + 1
    npad = rows * 128
    ucl = jnp.zeros((npad,), jnp.int32).at[uidx].set(c)
    starts = jnp.full((npad,), n, jnp.int32).at[uidx].min(
        jnp.arange(n, dtype=jnp.int32))
    per = n // NW
    ju0 = uidx[0::per]
    nu = uidx[per - 1::per] - ju0 + 1
    return ucl.reshape(-1, 128), starts.reshape(-1, 128), ju0, nu


def kernel(user, item_i, item_j, embed_user_w, embed_item_w):
    iu = user.astype(jnp.int32)
    it = jnp.concatenate([item_i.astype(jnp.int32), item_j.astype(jnp.int32)])
    su, pu = lax.sort_key_val(iu, jnp.arange(NU, dtype=jnp.int32))
    st, pt = lax.sort_key_val(it, jnp.arange(NT, dtype=jnp.int32))
    inv_u = jnp.zeros((NU,), jnp.int32).at[pu].set(
        jnp.arange(NU, dtype=jnp.int32))
    inv_t = jnp.zeros((NT,), jnp.int32).at[pt].set(
        jnp.arange(NT, dtype=jnp.int32))

    uclu2, sttu2, ju0u, nuu = _prep(su, NU)
    uclt2, sttt2, ju0t, nut = _prep(st, NT)
    meta = jnp.zeros((8, 128), jnp.int32)
    meta = meta.at[0, :NW].set(ju0u).at[1, :NW].set(nuu)
    meta = meta.at[2, :NW].set(ju0t).at[3, :NW].set(nut)

    gu, gt = _stage1(su.reshape(-1, 128), st.reshape(-1, 128),
                     uclu2, sttu2, uclt2, sttt2, meta,
                     embed_user_w.T, embed_item_w.T)
    return _stage2(inv_u.reshape(-1, 128), inv_t.reshape(-1, 128), gu, gt)


# speedup vs baseline: 3.3064x; 2.6254x over previous
import functools

import jax
import jax.numpy as jnp
from jax import lax
from jax.experimental import pallas as pl
from jax.experimental.pallas import tpu as pltpu
from jax.experimental.pallas import tpu_sc as plsc

N_FACTORS = 64
BATCH = 16384
NU = BATCH
NT = 2 * BATCH
NC = 2
NS = 16
LANES = 16
NW = NC * NS
CHUNKS = N_FACTORS // LANES
NBLK = 7813
SBW = 4
NSB = (NBLK + SBW - 1) // SBW
SB_PER_W = (NSB + NW - 1) // NW
LAST_SB = NSB - 1
PCAP = 8192
MCAP = 8192


def _sc(ref, pos):
    return ref[pl.ds(pos, LANES)][0]


def _stage1_body(user2, itemi2, itemj2, ut, itt, gu3, gt3,
                 idx_v, rbuf, pbuf, mbuf, mpbuf, blk0, blk1, rowst,
                 sb0, sb1, so):
    wid = lax.axis_index("s") * NC + lax.axis_index("c")
    sb_lo = wid * SB_PER_W
    sb_hi = jnp.minimum(sb_lo + SB_PER_W, NSB)
    nq = sb_hi - sb_lo
    iota = lax.iota(jnp.int32, LANES)

    def scan(src2, pos_base, cnt0):
        pltpu.sync_copy(src2, idx_v)

        def chunk(k, cnt):
            rv = idx_v[k >> 3, pl.ds((k & 7) * LANES, LANES)]
            sv = rv >> 9
            m = (sv >= sb_lo) & (sv < sb_hi)
            plsc.store_compressed(rbuf.at[pl.ds(cnt, LANES)], rv, mask=m)
            pv = iota + (k * LANES + pos_base)
            plsc.store_compressed(pbuf.at[pl.ds(cnt, LANES)], pv, mask=m)
            pc = plsc.all_reduce_population_count(m)[0]
            return jnp.minimum(cnt + pc, PCAP - LANES)

        return lax.fori_loop(0, BATCH // LANES, chunk, cnt0)

    def fetch(tbl, sb, blk, sem):
        c0 = sb * 128 * SBW

        @pl.when(sb != LAST_SB)
        def _():
            pltpu.async_copy(
                tbl.at[:, pl.ds(pl.multiple_of(c0, 128), 128 * SBW)],
                blk, sem)

        @pl.when(sb == LAST_SB)
        def _():
            pltpu.async_copy(
                tbl.at[:, pl.ds(pl.multiple_of(c0, 128), 128)],
                blk.at[:, pl.ds(0, 128)], sem)

    def wait_fetch(tbl, sb, blk, sem):
        @pl.when(sb != LAST_SB)
        def _():
            pltpu.make_async_copy(
                tbl.at[:, pl.ds(0, 128 * SBW)], blk, sem).wait()

        @pl.when(sb == LAST_SB)
        def _():
            pltpu.make_async_copy(
                tbl.at[:, pl.ds(0, 128)], blk.at[:, pl.ds(0, 128)],
                sem).wait()

    def do_sb(sb, blk, npairs, gcnt, gout):
        def mchunk(k, mcnt):
            base = k * LANES
            rv = rbuf[pl.ds(base, LANES)]
            m = ((rv >> 9) == sb) & ((iota + base) < npairs)
            plsc.store_compressed(mbuf.at[pl.ds(mcnt, LANES)], rv, mask=m)
            pv = pbuf[pl.ds(base, LANES)]
            plsc.store_compressed(mpbuf.at[pl.ds(mcnt, LANES)], pv, mask=m)
            pc = plsc.all_reduce_population_count(m)[0]
            return jnp.minimum(mcnt + pc, MCAP - LANES)

        nchunks = (npairs + LANES - 1) // LANES
        mcnt = lax.fori_loop(0, nchunks, mchunk, 0)

        def ext(t, g):
            r = _sc(mbuf, t)
            pos = _sc(mpbuf, t)
            l = (r & 127) + 128 * ((r >> 7) & (SBW - 1))
            slot = g & 7

            @pl.when(g >= 8)
            def _():
                pltpu.make_async_copy(
                    rowst.at[0], gout.at[0], so).wait()
            for kk in range(CHUNKS):
                gth = plsc.load_gather(
                    blk, [iota + kk * LANES, jnp.full((LANES,), l, jnp.int32)])
                rowst[slot, 0, pl.ds(kk * LANES, LANES)] = gth
            pltpu.async_copy(rowst.at[slot], gout.at[pos], so)
            return g + 1

        return lax.fori_loop(0, mcnt, ext, gcnt)

    def run_pass(tbl, npairs, gout):
        def qbody(q2, gcnt):
            q = 2 * q2
            sb = sb_lo + q
            wait_fetch(tbl, sb, blk0, sb0)

            @pl.when(q + 1 < nq)
            def _():
                fetch(tbl, sb + 1, blk1, sb1)
            gcnt = do_sb(sb, blk0, npairs, gcnt, gout)

            def second(gc):
                wait_fetch(tbl, sb + 1, blk1, sb1)

                @pl.when(q + 2 < nq)
                def _():
                    fetch(tbl, sb + 2, blk0, sb0)
                return do_sb(sb + 1, blk1, npairs, gc, gout)

            gcnt = lax.cond(q + 1 < nq, second, lambda gc: gc, gcnt)
            return gcnt

        fetch(tbl, sb_lo, blk0, sb0)
        gcnt = lax.fori_loop(0, (nq + 1) >> 1, qbody, 0)

        def drain(i, c):
            pltpu.make_async_copy(rowst.at[0], gout.at[0], so).wait()
            return c

        lax.fori_loop(0, jnp.minimum(gcnt, 8), drain, 0)

    nu_pairs = scan(user2, 0, 0)
    run_pass(ut, nu_pairs, gu3)
    nt_pairs = scan(itemi2, 0, 0)
    nt_pairs = scan(itemj2, BATCH, nt_pairs)
    run_pass(itt, nt_pairs, gt3)


@functools.partial(
    pl.kernel,
    mesh=plsc.VectorSubcoreMesh(core_axis_name="c", subcore_axis_name="s"),
    out_type=(jax.ShapeDtypeStruct((NU, 1, 128), jnp.float32),
              jax.ShapeDtypeStruct((NT, 1, 128), jnp.float32)),
    compiler_params=pltpu.CompilerParams(needs_layout_passes=False),
    scratch_types=[
        pltpu.VMEM((128, 128), jnp.int32),
        pltpu.VMEM((PCAP,), jnp.int32),
        pltpu.VMEM((PCAP,), jnp.int32),
        pltpu.VMEM((MCAP,), jnp.int32),
        pltpu.VMEM((MCAP,), jnp.int32),
        pltpu.VMEM((64, 128 * SBW), jnp.float32),
        pltpu.VMEM((64, 128 * SBW), jnp.float32),
        pltpu.VMEM((8, 1, 128), jnp.float32),
        pltpu.SemaphoreType.DMA,
        pltpu.SemaphoreType.DMA,
        pltpu.SemaphoreType.DMA,
    ],
)
def _stage1(*args):
    _stage1_body(*args)


def _stage2_body(gu3, gt3, out, u3, vi3, vj3, out_v, sem):
    wid = lax.axis_index("s") * NC + lax.axis_index("c")
    lane = lax.iota(jnp.int32, LANES)
    for h in range(2):
        base = wid * 512 + h * 256
        cps = [pltpu.async_copy(gu3.at[pl.ds(base, 256)], u3, sem),
               pltpu.async_copy(gt3.at[pl.ds(base, 256)], vi3, sem),
               pltpu.async_copy(gt3.at[pl.ds(BATCH + base, 256)], vj3, sem)]
        for cp in cps:
            cp.wait()

        def group_body(g, carry):
            gb = g * LANES
            res = jnp.zeros((LANES,), jnp.float32)
            for i in range(LANES):
                r = gb + i
                acc = jnp.zeros((LANES,), jnp.float32)
                for c in range(CHUNKS):
                    sl = pl.ds(c * LANES, LANES)
                    acc = acc + u3[r, 0, sl] * (vi3[r, 0, sl] - vj3[r, 0, sl])
                res = jnp.where(lane == i, jnp.sum(acc), res)
            out_v[pl.ds(gb, LANES)] = res
            return carry

        lax.fori_loop(0, 256 // LANES, group_body, 0)
        pltpu.sync_copy(out_v, out.at[pl.ds(base, 256)])


@functools.partial(
    pl.kernel,
    mesh=plsc.VectorSubcoreMesh(core_axis_name="c", subcore_axis_name="s"),
    out_type=jax.ShapeDtypeStruct((BATCH,), jnp.float32),
    compiler_params=pltpu.CompilerParams(needs_layout_passes=False),
    scratch_types=[
        pltpu.VMEM((256, 1, 128), jnp.float32),
        pltpu.VMEM((256, 1, 128), jnp.float32),
        pltpu.VMEM((256, 1, 128), jnp.float32),
        pltpu.VMEM((256,), jnp.float32),
        pltpu.SemaphoreType.DMA,
    ],
)
def _stage2(*args):
    _stage2_body(*args)


def kernel(user, item_i, item_j, embed_user_w, embed_item_w):
    user2 = user.astype(jnp.int32).reshape(128, 128)
    itemi2 = item_i.astype(jnp.int32).reshape(128, 128)
    itemj2 = item_j.astype(jnp.int32).reshape(128, 128)
    gu3, gt3 = _stage1(user2, itemi2, itemj2,
                       embed_user_w.T, embed_item_w.T)
    return _stage2(gu3, gt3)


# speedup vs baseline: 3.7733x; 1.1412x over previous
import functools

import jax
import jax.numpy as jnp
from jax import lax
from jax.experimental import pallas as pl
from jax.experimental.pallas import tpu as pltpu
from jax.experimental.pallas import tpu_sc as plsc

N_FACTORS = 64
BATCH = 16384
NU = BATCH
NT = 2 * BATCH
NC = 2
NS = 16
LANES = 16
NW = NC * NS
CHUNKS = N_FACTORS // LANES
NBLK = 7813
SBW = 4
NSB = (NBLK + SBW - 1) // SBW
SB_PER_W = (NSB + NW - 1) // NW
LAST_SB = NSB - 1
PCAP = 4096
MCAP = 4096


def _sc(ref, pos):
    return ref[pl.ds(pos, LANES)][0]


def _stage1_body(user2, itemi2, itemj2, ut, itt, gu3, gt3,
                 idx_v, rbuf, pbuf, mbuf, mpbuf, blk0, blk1, blk2, rowst,
                 sb0, sb1, sb2, so):
    wid = lax.axis_index("s") * NC + lax.axis_index("c")
    sb_lo = wid * SB_PER_W
    sb_hi = jnp.minimum(sb_lo + SB_PER_W, NSB)
    nq = sb_hi - sb_lo
    iota = lax.iota(jnp.int32, LANES)

    def scan(src2, pos_base, cnt0):
        cnt = cnt0
        for hh in range(2):
            pltpu.sync_copy(src2.at[pl.ds(hh * 64, 64)], idx_v)

            def chunk(k, cnt, _hh=hh):
                rv = idx_v[k >> 3, pl.ds((k & 7) * LANES, LANES)]
                sv = rv >> 9
                m = (sv >= sb_lo) & (sv < sb_hi)
                plsc.store_compressed(rbuf.at[pl.ds(cnt, LANES)], rv, mask=m)
                pv = iota + (k * LANES + (pos_base + _hh * 8192))
                plsc.store_compressed(pbuf.at[pl.ds(cnt, LANES)], pv, mask=m)
                pc = plsc.all_reduce_population_count(m)[0]
                return jnp.minimum(cnt + pc, PCAP - LANES)

            cnt = lax.fori_loop(0, BATCH // LANES // 2, chunk, cnt)
        return cnt

    def fetch(tbl, sb, blk, sem):
        c0 = sb * 128 * SBW

        @pl.when(sb != LAST_SB)
        def _():
            pltpu.async_copy(
                tbl.at[:, pl.ds(pl.multiple_of(c0, 128), 128 * SBW)],
                blk, sem)

        @pl.when(sb == LAST_SB)
        def _():
            pltpu.async_copy(
                tbl.at[:, pl.ds(pl.multiple_of(c0, 128), 128)],
                blk.at[:, pl.ds(0, 128)], sem)

    def wait_fetch(tbl, sb, blk, sem):
        @pl.when(sb != LAST_SB)
        def _():
            pltpu.make_async_copy(
                tbl.at[:, pl.ds(0, 128 * SBW)], blk, sem).wait()

        @pl.when(sb == LAST_SB)
        def _():
            pltpu.make_async_copy(
                tbl.at[:, pl.ds(0, 128)], blk.at[:, pl.ds(0, 128)],
                sem).wait()

    def do_sb(sb, blk, npairs, gcnt, gout):
        def mchunk(k, mcnt):
            base = k * LANES
            rv = rbuf[pl.ds(base, LANES)]
            m = (rv >> 9) == sb
            plsc.store_compressed(mbuf.at[pl.ds(mcnt, LANES)], rv, mask=m)
            pv = pbuf[pl.ds(base, LANES)]
            plsc.store_compressed(mpbuf.at[pl.ds(mcnt, LANES)], pv, mask=m)
            pc = plsc.all_reduce_population_count(m)[0]
            return jnp.minimum(mcnt + pc, MCAP - LANES)

        nchunks = (npairs + LANES - 1) // LANES
        mcnt = lax.fori_loop(0, nchunks, mchunk, 0)

        def ext(t, g):
            r = _sc(mbuf, t)
            pos = _sc(mpbuf, t)
            l = (r & 127) + 128 * ((r >> 7) & (SBW - 1))
            slot = g & 7

            @pl.when(g >= 8)
            def _():
                pltpu.make_async_copy(
                    rowst.at[0], gout.at[0], so).wait()
            for kk in range(CHUNKS):
                gth = plsc.load_gather(
                    blk, [iota + kk * LANES, jnp.full((LANES,), l, jnp.int32)])
                rowst[slot, 0, pl.ds(kk * LANES, LANES)] = gth
            pltpu.async_copy(rowst.at[slot], gout.at[pos], so)
            return g + 1

        return lax.fori_loop(0, mcnt, ext, gcnt)

    def run_pass(tbl, npairs, gout):
        plsc.store_compressed(rbuf.at[pl.ds(npairs, LANES)],
                              jnp.full((LANES,), -1, jnp.int32),
                              mask=jnp.full((LANES,), True))

        bufs = (blk0, blk1, blk2)
        sems = (sb0, sb1, sb2)
        fetch(tbl, sb_lo, blk0, sb0)

        @pl.when(1 < nq)
        def _():
            fetch(tbl, sb_lo + 1, blk1, sb1)

        def qbody(q3, gcnt):
            q = 3 * q3
            for i in range(3):
                def step(gc, _i=i):
                    sb = sb_lo + q + _i
                    wait_fetch(tbl, sb, bufs[_i], sems[_i])

                    @pl.when(q + _i + 2 < nq)
                    def _():
                        fetch(tbl, sb + 2, bufs[(_i + 2) % 3],
                              sems[(_i + 2) % 3])
                    return do_sb(sb, bufs[_i], npairs, gc, gout)

                gcnt = lax.cond(q + i < nq, step, lambda gc: gc, gcnt)
            return gcnt

        gcnt = lax.fori_loop(0, (nq + 2) // 3, qbody, 0)

        def drain(i, c):
            pltpu.make_async_copy(rowst.at[0], gout.at[0], so).wait()
            return c

        lax.fori_loop(0, jnp.minimum(gcnt, 8), drain, 0)

    nu_pairs = scan(user2, 0, 0)
    run_pass(ut, nu_pairs, gu3)
    nt_pairs = scan(itemi2, 0, 0)
    nt_pairs = scan(itemj2, BATCH, nt_pairs)
    run_pass(itt, nt_pairs, gt3)


@functools.partial(
    pl.kernel,
    mesh=plsc.VectorSubcoreMesh(core_axis_name="c", subcore_axis_name="s"),
    out_type=(jax.ShapeDtypeStruct((NU, 1, 128), jnp.float32),
              jax.ShapeDtypeStruct((NT, 1, 128), jnp.float32)),
    compiler_params=pltpu.CompilerParams(needs_layout_passes=False),
    scratch_types=[
        pltpu.VMEM((64, 128), jnp.int32),
        pltpu.VMEM((PCAP,), jnp.int32),
        pltpu.VMEM((PCAP,), jnp.int32),
        pltpu.VMEM((MCAP,), jnp.int32),
        pltpu.VMEM((MCAP,), jnp.int32),
        pltpu.VMEM((64, 128 * SBW), jnp.float32),
        pltpu.VMEM((64, 128 * SBW), jnp.float32),
        pltpu.VMEM((64, 128 * SBW), jnp.float32),
        pltpu.VMEM((8, 1, 128), jnp.float32),
        pltpu.SemaphoreType.DMA,
        pltpu.SemaphoreType.DMA,
        pltpu.SemaphoreType.DMA,
        pltpu.SemaphoreType.DMA,
    ],
)
def _stage1(*args):
    _stage1_body(*args)


def _stage2_body(gu3, gt3, out, u3, vi3, vj3, out_v, sem):
    wid = lax.axis_index("s") * NC + lax.axis_index("c")
    lane = lax.iota(jnp.int32, LANES)
    for h in range(2):
        base = wid * 512 + h * 256
        cps = [pltpu.async_copy(gu3.at[pl.ds(base, 256)], u3, sem),
               pltpu.async_copy(gt3.at[pl.ds(base, 256)], vi3, sem),
               pltpu.async_copy(gt3.at[pl.ds(BATCH + base, 256)], vj3, sem)]
        for cp in cps:
            cp.wait()

        def group_body(g, carry):
            gb = g * LANES
            res = jnp.zeros((LANES,), jnp.float32)
            for i in range(LANES):
                r = gb + i
                acc = jnp.zeros((LANES,), jnp.float32)
                for c in range(CHUNKS):
                    sl = pl.ds(c * LANES, LANES)
                    acc = acc + u3[r, 0, sl] * (vi3[r, 0, sl] - vj3[r, 0, sl])
                res = jnp.where(lane == i, jnp.sum(acc), res)
            out_v[pl.ds(gb, LANES)] = res
            return carry

        lax.fori_loop(0, 256 // LANES, group_body, 0)
        pltpu.sync_copy(out_v, out.at[pl.ds(base, 256)])


@functools.partial(
    pl.kernel,
    mesh=plsc.VectorSubcoreMesh(core_axis_name="c", subcore_axis_name="s"),
    out_type=jax.ShapeDtypeStruct((BATCH,), jnp.float32),
    compiler_params=pltpu.CompilerParams(needs_layout_passes=False),
    scratch_types=[
        pltpu.VMEM((256, 1, 128), jnp.float32),
        pltpu.VMEM((256, 1, 128), jnp.float32),
        pltpu.VMEM((256, 1, 128), jnp.float32),
        pltpu.VMEM((256,), jnp.float32),
        pltpu.SemaphoreType.DMA,
    ],
)
def _stage2(*args):
    _stage2_body(*args)


def kernel(user, item_i, item_j, embed_user_w, embed_item_w):
    user2 = user.astype(jnp.int32).reshape(128, 128)
    itemi2 = item_i.astype(jnp.int32).reshape(128, 128)
    itemj2 = item_j.astype(jnp.int32).reshape(128, 128)
    gu3, gt3 = _stage1(user2, itemi2, itemj2,
                       embed_user_w.T, embed_item_w.T)
    return _stage2(gu3, gt3)
